# baseline (speedup 1.0000x reference)
import jax
import jax.numpy as jnp
from jax import lax
from jax.experimental import pallas as pl
from jax.experimental.pallas import tpu as pltpu
from jax.experimental.pallas import tpu_sc as plsc

N = 10000
E = 320000
D_IN = 128
H = 64
NC, NS = 2, 16
NW = NC * NS
EPW = E // NW
CH = 125
NCH = EPW // CH
N_PAD = 10240
RPT = N_PAD // NS
D0 = H + 8
NBUF = 4
K_AHEAD = 2


def _make_edge_agg(d, with_deg):
    mesh = plsc.VectorSubcoreMesh(
        core_axis_name="c", subcore_axis_name="s", num_cores=NC, num_subcores=NS)
    K_AHEAD = 3 if d == 8 else 2

    def body(*refs):
        if with_deg:
            (p_hbm, src_hbm, dst_hbm, zero_hbm, zero1_hbm, ones_hbm,
             out_hbm, deg_hbm, srcv, dstv) = refs[:10]
            bufs = refs[10:10 + NBUF]
            onesv, accsh, degsh = refs[10 + NBUF:13 + NBUF]
            gsem = refs[13 + NBUF:13 + 2 * NBUF]
            ssem = refs[13 + 2 * NBUF:13 + 3 * NBUF]
            isem, zsem, dsem = refs[13 + 3 * NBUF:]
        else:
            (p_hbm, src_hbm, dst_hbm, zero_hbm, out_hbm,
             srcv, dstv) = refs[:7]
            bufs = refs[7:7 + NBUF]
            accsh = refs[7 + NBUF]
            gsem = refs[8 + NBUF:8 + 2 * NBUF]
            ssem = refs[8 + 2 * NBUF:8 + 3 * NBUF]
            isem, zsem = refs[8 + 3 * NBUF:]
        c = lax.axis_index("c")
        s = lax.axis_index("s")
        w = c * NS + s
        r0 = s * RPT
        zcp = pltpu.async_copy(
            zero_hbm.at[pl.ds(r0, RPT)], accsh.at[pl.ds(r0, RPT)], zsem)
        if with_deg:
            zcp1 = pltpu.async_copy(
                zero1_hbm.at[pl.ds(r0, RPT)], degsh.at[pl.ds(r0, RPT)], zsem)
            ocp = pltpu.async_copy(ones_hbm, onesv, isem)
        scp = pltpu.async_copy(src_hbm.at[w], srcv, isem)
        dcp = pltpu.async_copy(dst_hbm.at[w], dstv, isem)
        scp.wait()
        dcp.wait()
        if with_deg:
            ocp.wait()
        for b in range(K_AHEAD):
            pltpu.async_copy(p_hbm.at[srcv.at[b]], bufs[b], gsem[b])
        zcp.wait()
        if with_deg:
            zcp1.wait()
        plsc.subcore_barrier()

        def round_(tt, carry):
            for b in range(NBUF):
                j = tt * NBUF + b
                bn = (b + K_AHEAD) % NBUF

                @pl.when(jnp.logical_and(j + K_AHEAD < NCH, j >= NBUF - K_AHEAD))
                def _():
                    pltpu.make_async_copy(
                        bufs[bn], accsh.at[dstv.at[j]], ssem[bn]).wait()

                @pl.when(j + K_AHEAD < NCH)
                def _():
                    pltpu.async_copy(
                        p_hbm.at[srcv.at[j + K_AHEAD]], bufs[bn],
                        gsem[bn])

                pltpu.make_async_copy(
                    p_hbm.at[srcv.at[j]], bufs[b], gsem[b]).wait()
                pltpu.async_copy(
                    bufs[b], accsh.at[dstv.at[j]], ssem[b], add=True)
                if with_deg:
                    @pl.when(j >= NBUF)
                    def _():
                        pltpu.make_async_copy(
                            onesv, degsh.at[dstv.at[j]], dsem).wait()

                    pltpu.async_copy(
                        onesv, degsh.at[dstv.at[j]], dsem, add=True)
            return carry

        lax.fori_loop(0, NCH // NBUF, round_, 0)
        for b in range(NBUF):
            pltpu.make_async_copy(
                bufs[b], accsh.at[dstv.at[0]], ssem[b]).wait()
            if with_deg:
                pltpu.make_async_copy(
                    onesv, degsh.at[dstv.at[0]], dsem).wait()
        plsc.subcore_barrier()
        pltpu.sync_copy(accsh.at[pl.ds(r0, RPT)], out_hbm.at[c, pl.ds(r0, RPT)])
        if with_deg:
            pltpu.sync_copy(
                degsh.at[pl.ds(r0, RPT)], deg_hbm.at[c, pl.ds(r0, RPT)])

    out_type = [jax.ShapeDtypeStruct((NC, N_PAD, d), jnp.float32)]
    bufs_scratch = [pltpu.VMEM((CH, d), jnp.float32) for _ in range(NBUF)]
    sems_scratch = [pltpu.SemaphoreType.DMA for _ in range(2 * NBUF)]
    if with_deg:
        out_type.append(jax.ShapeDtypeStruct((NC, N_PAD, 1), jnp.float32))
        scratch = (
            [pltpu.VMEM((NCH, CH), jnp.int32)] * 2
            + bufs_scratch
            + [pltpu.VMEM((CH, 1), jnp.float32),
               pltpu.VMEM_SHARED((N_PAD, d), jnp.float32),
               pltpu.VMEM_SHARED((N_PAD, 1), jnp.float32)]
            + sems_scratch
            + [pltpu.SemaphoreType.DMA] * 3)
    else:
        scratch = (
            [pltpu.VMEM((NCH, CH), jnp.int32)] * 2
            + bufs_scratch
            + [pltpu.VMEM_SHARED((N_PAD, d), jnp.float32)]
            + sems_scratch
            + [pltpu.SemaphoreType.DMA] * 2)
    return pl.kernel(
        body,
        out_type=out_type,
        mesh=mesh,
        scratch_types=scratch,
        compiler_params=pltpu.CompilerParams(use_tc_tiling_on_sc=False),
    )


_agg_cache = {}


def _edge_agg(d):
    if d not in _agg_cache:
        _agg_cache[d] = _make_edge_agg(d, False)
    return _agg_cache[d]


BLK = 2000
NBLK = N // BLK


def _row_spec(*dims):
    if len(dims) == 1:
        return pl.BlockSpec((BLK, dims[0]), lambda i: (i, 0))
    return pl.BlockSpec((dims[0], BLK, dims[1]), lambda i: (0, i, 0))


def _full_spec(*shape):
    return pl.BlockSpec(shape, lambda i: tuple(0 for _ in shape))


def _mm0p_body(x_ref, w_ref, p_ref):
    p_ref[:, :H] = jnp.dot(
        x_ref[...], w_ref[...], preferred_element_type=jnp.float32)
    p_ref[:, H:] = jnp.ones((BLK, 8), jnp.float32)


def _mm0s_body(x_ref, w_ref, b_ref, s_ref):
    s_ref[...] = jnp.dot(
        x_ref[...], w_ref[...], preferred_element_type=jnp.float32) + b_ref[...]


def _cmb1p_body(s_ref, acc_ref, w_ref, h_out, p_out, inv_ref):
    agg = acc_ref[0, :, :H] + acc_ref[1, :, :H]
    deg = acc_ref[0, :, H:H + 1] + acc_ref[1, :, H:H + 1]
    inv = 1.0 / jnp.maximum(deg, 1.0)
    inv_ref[...] = inv
    h = jnp.maximum(s_ref[...] + agg * inv, 0.0)
    h_out[...] = h
    p_out[...] = jnp.dot(h, w_ref[...], preferred_element_type=jnp.float32)


def _cmbp_body(s_ref, acc_ref, inv_ref, w_ref, h_out, p_out):
    agg = acc_ref[0, :, :] + acc_ref[1, :, :]
    h = jnp.maximum(s_ref[...] + agg * inv_ref[...], 0.0)
    h_out[...] = h
    p_out[...] = jnp.dot(h, w_ref[...], preferred_element_type=jnp.float32)


def _s_body(h_ref, w_ref, b_ref, s_out):
    s_out[...] = jnp.dot(
        h_ref[...], w_ref[...], preferred_element_type=jnp.float32) + b_ref[...]


def _fin_body(s_ref, acc_ref, inv_ref, o_ref):
    agg = acc_ref[0, :, 0:1] + acc_ref[1, :, 0:1]
    o_ref[...] = s_ref[:, 0:1] + agg * inv_ref[...]


def _f32(*shape):
    return jax.ShapeDtypeStruct(shape, jnp.float32)


def kernel(x, edge_index, Ws, Wn, bs):
    src = edge_index[0].astype(jnp.int32).reshape(NW, NCH, CH)
    dst = edge_index[1].astype(jnp.int32).reshape(NW, NCH, CH)
    zH = jnp.zeros((N_PAD, H), jnp.float32)
    z0 = jnp.zeros((N_PAD, D0), jnp.float32)
    z8 = jnp.zeros((N_PAD, 8), jnp.float32)

    pad7 = ((0, 0), (0, 7))
    wn3 = jnp.pad(Wn[3], pad7)
    ws3 = jnp.pad(Ws[3], pad7)
    b3 = jnp.pad(bs[3], (0, 7))

    p0 = pl.pallas_call(
        _mm0p_body, grid=(NBLK,),
        in_specs=[_row_spec(D_IN), _full_spec(D_IN, H)],
        out_specs=_row_spec(D0),
        out_shape=_f32(N, D0))(x, Wn[0])
    acc0 = _edge_agg(D0)(p0, src, dst, z0)[0]
    s0 = pl.pallas_call(
        _mm0s_body, grid=(NBLK,),
        in_specs=[_row_spec(D_IN), _full_spec(D_IN, H), _full_spec(1, H)],
        out_specs=_row_spec(H),
        out_shape=_f32(N, H))(x, Ws[0], bs[0][None, :])

    h1, p1, inv = pl.pallas_call(
        _cmb1p_body, grid=(NBLK,),
        in_specs=[_row_spec(H), _row_spec(NC, D0), _full_spec(H, H)],
        out_specs=[_row_spec(H), _row_spec(H), _row_spec(1)],
        out_shape=[_f32(N, H), _f32(N, H), _f32(N, 1)])(s0, acc0, Wn[1])
    acc1 = _edge_agg(H)(p1, src, dst, zH)[0]
    s1 = pl.pallas_call(
        _s_body, grid=(NBLK,),
        in_specs=[_row_spec(H), _full_spec(H, H), _full_spec(1, H)],
        out_specs=_row_spec(H),
        out_shape=_f32(N, H))(h1, Ws[1], bs[1][None, :])

    h2, p2 = pl.pallas_call(
        _cmbp_body, grid=(NBLK,),
        in_specs=[_row_spec(H), _row_spec(NC, H), _row_spec(1),
                  _full_spec(H, H)],
        out_specs=[_row_spec(H), _row_spec(H)],
        out_shape=[_f32(N, H), _f32(N, H)])(s1, acc1, inv, Wn[2])
    acc2 = _edge_agg(H)(p2, src, dst, zH)[0]
    s2 = pl.pallas_call(
        _s_body, grid=(NBLK,),
        in_specs=[_row_spec(H), _full_spec(H, H), _full_spec(1, H)],
        out_specs=_row_spec(H),
        out_shape=_f32(N, H))(h2, Ws[2], bs[2][None, :])

    h3, p3 = pl.pallas_call(
        _cmbp_body, grid=(NBLK,),
        in_specs=[_row_spec(H), _row_spec(NC, H), _row_spec(1),
                  _full_spec(H, 8)],
        out_specs=[_row_spec(H), _row_spec(8)],
        out_shape=[_f32(N, H), _f32(N, 8)])(s2, acc2, inv, wn3)
    acc3 = _edge_agg(8)(p3, src, dst, z8)[0]
    s3 = pl.pallas_call(
        _s_body, grid=(NBLK,),
        in_specs=[_row_spec(H), _full_spec(H, 8), _full_spec(1, 8)],
        out_specs=_row_spec(8),
        out_shape=_f32(N, 8))(h3, ws3, b3[None, :])

    out = pl.pallas_call(
        _fin_body, grid=(NBLK,),
        in_specs=[_row_spec(8), _row_spec(NC, 8), _row_spec(1)],
        out_specs=_row_spec(1),
        out_shape=_f32(N, 1))(s3, acc3, inv)
    return jnp.squeeze(out, axis=-1)


# speedup vs baseline: 1.0082x; 1.0082x over previous
import jax
import jax.numpy as jnp
from jax import lax
from jax.experimental import pallas as pl
from jax.experimental.pallas import tpu as pltpu
from jax.experimental.pallas import tpu_sc as plsc

N = 10000
E = 320000
D_IN = 128
H = 64
NC, NS = 2, 16
NW = NC * NS
EPW = E // NW
CH = 125
NCH = EPW // CH
N_PAD = 10240
RPT = N_PAD // NS
D0 = H + 8
NBUF = 4
K_AHEAD = 2


def _make_edge_agg(d, with_deg):
    mesh = plsc.VectorSubcoreMesh(
        core_axis_name="c", subcore_axis_name="s", num_cores=NC, num_subcores=NS)
    K_AHEAD = 3 if d == 8 else 2

    def body(*refs):
        if with_deg:
            (p_hbm, src_hbm, dst_hbm, zero_hbm, zero1_hbm, ones_hbm,
             out_hbm, deg_hbm, srcv, dstv) = refs[:10]
            bufs = refs[10:10 + NBUF]
            onesv, accsh, degsh = refs[10 + NBUF:13 + NBUF]
            gsem = refs[13 + NBUF:13 + 2 * NBUF]
            ssem = refs[13 + 2 * NBUF:13 + 3 * NBUF]
            isem, zsem, dsem = refs[13 + 3 * NBUF:]
        else:
            (p_hbm, src_hbm, dst_hbm, zero_hbm, out_hbm,
             srcv, dstv) = refs[:7]
            bufs = refs[7:7 + NBUF]
            accsh = refs[7 + NBUF]
            gsem = refs[8 + NBUF:8 + 2 * NBUF]
            ssem = refs[8 + 2 * NBUF:8 + 3 * NBUF]
            isem, zsem = refs[8 + 3 * NBUF:]
        c = lax.axis_index("c")
        s = lax.axis_index("s")
        w = c * NS + s
        r0 = s * RPT
        zcp = pltpu.async_copy(
            zero_hbm.at[pl.ds(r0, RPT)], accsh.at[pl.ds(r0, RPT)], zsem)
        if with_deg:
            zcp1 = pltpu.async_copy(
                zero1_hbm.at[pl.ds(r0, RPT)], degsh.at[pl.ds(r0, RPT)], zsem)
            ocp = pltpu.async_copy(ones_hbm, onesv, isem)
        scp = pltpu.async_copy(src_hbm.at[w], srcv, isem)
        dcp = pltpu.async_copy(dst_hbm.at[w], dstv, isem)
        scp.wait()
        dcp.wait()
        if with_deg:
            ocp.wait()
        for b in range(K_AHEAD):
            pltpu.async_copy(p_hbm.at[srcv.at[b]], bufs[b], gsem[b])
        zcp.wait()
        if with_deg:
            zcp1.wait()
        plsc.subcore_barrier()

        def round_(tt, carry):
            for b in range(NBUF):
                j = tt * NBUF + b
                bn = (b + K_AHEAD) % NBUF

                @pl.when(jnp.logical_and(j + K_AHEAD < NCH, j >= NBUF - K_AHEAD))
                def _():
                    pltpu.make_async_copy(
                        bufs[bn], accsh.at[dstv.at[j]], ssem[bn]).wait()

                @pl.when(j + K_AHEAD < NCH)
                def _():
                    pltpu.async_copy(
                        p_hbm.at[srcv.at[j + K_AHEAD]], bufs[bn],
                        gsem[bn])

                pltpu.make_async_copy(
                    p_hbm.at[srcv.at[j]], bufs[b], gsem[b]).wait()
                pltpu.async_copy(
                    bufs[b], accsh.at[dstv.at[j]], ssem[b], add=True)
                if with_deg:
                    @pl.when(j >= NBUF)
                    def _():
                        pltpu.make_async_copy(
                            onesv, degsh.at[dstv.at[j]], dsem).wait()

                    pltpu.async_copy(
                        onesv, degsh.at[dstv.at[j]], dsem, add=True)
            return carry

        lax.fori_loop(0, NCH // NBUF, round_, 0)
        for b in range(NBUF):
            pltpu.make_async_copy(
                bufs[b], accsh.at[dstv.at[0]], ssem[b]).wait()
            if with_deg:
                pltpu.make_async_copy(
                    onesv, degsh.at[dstv.at[0]], dsem).wait()
        plsc.subcore_barrier()
        pltpu.sync_copy(accsh.at[pl.ds(r0, RPT)], out_hbm.at[c, pl.ds(r0, RPT)])
        if with_deg:
            pltpu.sync_copy(
                degsh.at[pl.ds(r0, RPT)], deg_hbm.at[c, pl.ds(r0, RPT)])

    out_type = [jax.ShapeDtypeStruct((NC, N_PAD, d), jnp.float32)]
    bufs_scratch = [pltpu.VMEM((CH, d), jnp.float32) for _ in range(NBUF)]
    sems_scratch = [pltpu.SemaphoreType.DMA for _ in range(2 * NBUF)]
    if with_deg:
        out_type.append(jax.ShapeDtypeStruct((NC, N_PAD, 1), jnp.float32))
        scratch = (
            [pltpu.VMEM((NCH, CH), jnp.int32)] * 2
            + bufs_scratch
            + [pltpu.VMEM((CH, 1), jnp.float32),
               pltpu.VMEM_SHARED((N_PAD, d), jnp.float32),
               pltpu.VMEM_SHARED((N_PAD, 1), jnp.float32)]
            + sems_scratch
            + [pltpu.SemaphoreType.DMA] * 3)
    else:
        scratch = (
            [pltpu.VMEM((NCH, CH), jnp.int32)] * 2
            + bufs_scratch
            + [pltpu.VMEM_SHARED((N_PAD, d), jnp.float32)]
            + sems_scratch
            + [pltpu.SemaphoreType.DMA] * 2)
    return pl.kernel(
        body,
        out_type=out_type,
        mesh=mesh,
        scratch_types=scratch,
        compiler_params=pltpu.CompilerParams(use_tc_tiling_on_sc=False),
    )


_agg_cache = {}


def _edge_agg(d):
    if d not in _agg_cache:
        _agg_cache[d] = _make_edge_agg(d, False)
    return _agg_cache[d]


BLK = 2000
NBLK = N // BLK


def _row_spec(*dims):
    if len(dims) == 1:
        return pl.BlockSpec((BLK, dims[0]), lambda i: (i, 0))
    return pl.BlockSpec((dims[0], BLK, dims[1]), lambda i: (0, i, 0))


def _full_spec(*shape):
    return pl.BlockSpec(shape, lambda i: tuple(0 for _ in shape))


def _mm0_body(x_ref, w_ref, b_ref, s_ref, p_ref):
    ps = jnp.dot(x_ref[...], w_ref[...], preferred_element_type=jnp.float32)
    s_ref[...] = ps[:, :H] + b_ref[...]
    p_ref[:, :H] = ps[:, H:]
    p_ref[:, H:] = jnp.ones((BLK, 8), jnp.float32)


def _cmb1_body(s_ref, acc_ref, w_ref, b_ref, s_out, p_out, inv_ref):
    agg = acc_ref[0, :, :H] + acc_ref[1, :, :H]
    deg = acc_ref[0, :, H:H + 1] + acc_ref[1, :, H:H + 1]
    inv = 1.0 / jnp.maximum(deg, 1.0)
    inv_ref[...] = inv
    h = jnp.maximum(s_ref[...] + agg * inv, 0.0)
    ps = jnp.dot(h, w_ref[...], preferred_element_type=jnp.float32)
    s_out[...] = ps[:, :H] + b_ref[...]
    p_out[...] = ps[:, H:]


def _cmb2_body(s_ref, acc_ref, inv_ref, w_ref, b_ref, s_out, p_out):
    agg = acc_ref[0, :, :] + acc_ref[1, :, :]
    h = jnp.maximum(s_ref[...] + agg * inv_ref[...], 0.0)
    ps = jnp.dot(h, w_ref[...], preferred_element_type=jnp.float32)
    s_out[...] = ps[:, :H] + b_ref[...]
    p_out[...] = ps[:, H:]


def _cmb3_body(s_ref, acc_ref, inv_ref, w_ref, b_ref, s_out, p_out):
    agg = acc_ref[0, :, :] + acc_ref[1, :, :]
    h = jnp.maximum(s_ref[...] + agg * inv_ref[...], 0.0)
    ps = jnp.dot(h, w_ref[...], preferred_element_type=jnp.float32)
    s_out[...] = ps[:, :8] + b_ref[...]
    p_out[...] = ps[:, 8:]


def _fin_body(s_ref, acc_ref, inv_ref, o_ref):
    agg = acc_ref[0, :, 0:1] + acc_ref[1, :, 0:1]
    o_ref[...] = s_ref[:, 0:1] + agg * inv_ref[...]


def _f32(*shape):
    return jax.ShapeDtypeStruct(shape, jnp.float32)


def kernel(x, edge_index, Ws, Wn, bs):
    src = edge_index[0].astype(jnp.int32).reshape(NW, NCH, CH)
    dst = edge_index[1].astype(jnp.int32).reshape(NW, NCH, CH)
    zH = jnp.zeros((N_PAD, H), jnp.float32)
    z0 = jnp.zeros((N_PAD, D0), jnp.float32)
    z8 = jnp.zeros((N_PAD, 8), jnp.float32)

    w0 = jnp.concatenate([Ws[0], Wn[0]], axis=1)
    w1 = jnp.concatenate([Ws[1], Wn[1]], axis=1)
    w2 = jnp.concatenate([Ws[2], Wn[2]], axis=1)
    pad7 = ((0, 0), (0, 7))
    w3 = jnp.concatenate([jnp.pad(Ws[3], pad7), jnp.pad(Wn[3], pad7)], axis=1)
    b3 = jnp.pad(bs[3], (0, 7))

    s0, p0 = pl.pallas_call(
        _mm0_body, grid=(NBLK,),
        in_specs=[_row_spec(D_IN), _full_spec(D_IN, 2 * H), _full_spec(1, H)],
        out_specs=[_row_spec(H), _row_spec(D0)],
        out_shape=[_f32(N, H), _f32(N, D0)])(x, w0, bs[0][None, :])
    acc0, = _edge_agg(D0)(p0, src, dst, z0)

    s1, p1, inv = pl.pallas_call(
        _cmb1_body, grid=(NBLK,),
        in_specs=[_row_spec(H), _row_spec(NC, D0), _full_spec(H, 2 * H),
                  _full_spec(1, H)],
        out_specs=[_row_spec(H), _row_spec(H), _row_spec(1)],
        out_shape=[_f32(N, H), _f32(N, H), _f32(N, 1)])(
            s0, acc0, w1, bs[1][None, :])
    acc1, = _edge_agg(H)(p1, src, dst, zH)

    s2, p2 = pl.pallas_call(
        _cmb2_body, grid=(NBLK,),
        in_specs=[_row_spec(H), _row_spec(NC, H), _row_spec(1),
                  _full_spec(H, 2 * H), _full_spec(1, H)],
        out_specs=[_row_spec(H), _row_spec(H)],
        out_shape=[_f32(N, H), _f32(N, H)])(
            s1, acc1, inv, w2, bs[2][None, :])
    acc2, = _edge_agg(H)(p2, src, dst, zH)

    s3, p3 = pl.pallas_call(
        _cmb3_body, grid=(NBLK,),
        in_specs=[_row_spec(H), _row_spec(NC, H), _row_spec(1),
                  _full_spec(H, 16), _full_spec(1, 8)],
        out_specs=[_row_spec(8), _row_spec(8)],
        out_shape=[_f32(N, 8), _f32(N, 8)])(
            s2, acc2, inv, w3, b3[None, :])
    acc3, = _edge_agg(8)(p3, src, dst, z8)

    out = pl.pallas_call(
        _fin_body, grid=(NBLK,),
        in_specs=[_row_spec(8), _row_spec(NC, 8), _row_spec(1)],
        out_specs=_row_spec(1),
        out_shape=_f32(N, 1))(s3, acc3, inv)
    return jnp.squeeze(out, axis=-1)
